# baseline (speedup 1.0000x reference)
import functools

import numpy as np
import jax
import jax.numpy as jnp
from jax import lax
from jax.experimental import pallas as pl
from jax.experimental.pallas import tpu as pltpu
from jax.experimental.pallas import tpu_sc as plsc

_B, _L, _D = 64, 2048, 64
_N = _B * _L

np.random.seed(0)
_MSHUF = np.random.permutation(np.arange(_N)).astype(np.int32)

_info = plsc.get_sparse_core_info()
_NC, _NS = _info.num_cores, _info.num_subcores
_NW = _NC * _NS
_RPW = _N // _NW
_LB = 256
_NT = _RPW // _LB
_TPB = _L // _LB

_mesh = plsc.VectorSubcoreMesh(core_axis_name="c", subcore_axis_name="s")
_PARAMS = pltpu.CompilerParams(use_tc_tiling_on_sc=False, needs_layout_passes=False)
_UNROLL = 16


@functools.partial(
    pl.kernel,
    mesh=_mesh,
    compiler_params=_PARAMS,
    out_type=jax.ShapeDtypeStruct((_N, _D), jnp.float32),
    scratch_types=[
        pltpu.VMEM((_NT, _LB), jnp.int32),
        pltpu.VMEM((8, 2, 8, 128), jnp.float32),
        pltpu.VMEM((8, 2, 8, 128), jnp.float32),
        pltpu.VMEM((_LB, _D), jnp.float32),
        pltpu.VMEM((_LB, _D), jnp.float32),
        pltpu.SemaphoreType.DMA,
        pltpu.SemaphoreType.DMA,
        pltpu.SemaphoreType.DMA,
        pltpu.SemaphoreType.DMA,
    ],
)
def _scatter_pass(x4_hbm, scat_hbm, out_hbm, sidx_v, blk0, blk1, rows0, rows1,
                  rsem0, rsem1, ssem0, ssem1):
    wid = lax.axis_index("s") * _NC + lax.axis_index("c")
    blks = (blk0, blk1)
    rows = (rows0, rows1)
    rsems = (rsem0, rsem1)
    ssems = (ssem0, ssem1)
    for t in range(_NT):
        pltpu.sync_copy(scat_hbm.at[pl.ds(wid * _RPW + t * _LB, _LB)],
                        sidx_v.at[t])
    col_ids = [jnp.arange(16, dtype=jnp.int32) + 16 * j for j in range(_LB // 16)]

    def read_block(t, s):
        b = 2 * wid + (t // _TPB)
        lt0 = 2 * (t % _TPB)
        return pltpu.async_copy(
            x4_hbm.at[pl.ds(b * 8, 8), pl.ds(lt0, 2), :, :], blks[s], rsems[s]
        )

    def transpose_block(s):
        blk, row = blks[s], rows[s]

        @plsc.parallel_loop(0, 8, 1)
        def body(dt):
            d8 = jnp.full((16,), dt * 8, jnp.int32)
            for di in range(8):
                d_splat = d8 + di
                for ltb in range(2):
                    for j in range(8):
                        v = blk[dt, ltb, di, pl.ds(16 * j, 16)]
                        plsc.store_scatter(
                            row, [col_ids[ltb * 8 + j], d_splat], v
                        )

    def scatter_block(t, s):
        return pltpu.async_copy(rows[s], out_hbm.at[sidx_v.at[t]], ssems[s])

    read_block(0, 0)
    read_block(1, 1)

    def pair_body(p, carry):
        for b in range(2):
            t = 2 * p + b
            pltpu.make_async_copy(
                x4_hbm.at[pl.ds(0, 8), pl.ds(0, 2), :, :], blks[b], rsems[b]
            ).wait()

            @pl.when(p > 0)
            def _():
                pltpu.make_async_copy(rows[b], out_hbm.at[pl.ds(0, _LB)],
                                      ssems[b]).wait()

            transpose_block(b)
            scatter_block(t, b)

            @pl.when(p < (_NT // 2 - 1))
            def _():
                read_block(t + 2, b)

        return carry

    lax.fori_loop(0, _NT // 2, pair_body, 0)
    pltpu.make_async_copy(rows[0], out_hbm.at[pl.ds(0, _LB)], ssems[0]).wait()
    pltpu.make_async_copy(rows[1], out_hbm.at[pl.ds(0, _LB)], ssems[1]).wait()


@functools.partial(
    pl.kernel,
    mesh=_mesh,
    compiler_params=_PARAMS,
    out_type=jax.ShapeDtypeStruct((512, 16, 8, 128), jnp.float32),
    scratch_types=[
        pltpu.VMEM((_LB, _D), jnp.float32),
        pltpu.VMEM((_LB, _D), jnp.float32),
        pltpu.VMEM((8, 2, 8, 128), jnp.float32),
        pltpu.VMEM((8, 2, 8, 128), jnp.float32),
        pltpu.SemaphoreType.DMA,
        pltpu.SemaphoreType.DMA,
        pltpu.SemaphoreType.DMA,
        pltpu.SemaphoreType.DMA,
    ],
)
def _untranspose_pass(src_hbm, out_hbm, rows0, rows1, blk0, blk1,
                      rsem0, rsem1, wsem0, wsem1):
    wid = lax.axis_index("s") * _NC + lax.axis_index("c")
    rows = (rows0, rows1)
    blks = (blk0, blk1)
    rsems = (rsem0, rsem1)
    wsems = (wsem0, wsem1)
    base = wid * _RPW
    col_ids = [jnp.arange(16, dtype=jnp.int32) + 16 * j for j in range(_LB // 16)]

    def read_block(t, s):
        return pltpu.async_copy(
            src_hbm.at[pl.ds(base + t * _LB, _LB)], rows[s], rsems[s]
        )

    def transpose_block(s):
        row, blk = rows[s], blks[s]

        @plsc.parallel_loop(0, 8, 1)
        def body(dt):
            d8 = jnp.full((16,), dt * 8, jnp.int32)
            for di in range(8):
                d_splat = d8 + di
                for ltb in range(2):
                    for j in range(8):
                        v = plsc.load_gather(
                            row, [col_ids[ltb * 8 + j], d_splat]
                        )
                        blk[dt, ltb, di, pl.ds(16 * j, 16)] = v

    def write_block(t, s):
        b = 2 * wid + (t // _TPB)
        lt0 = 2 * (t % _TPB)
        return pltpu.async_copy(
            blks[s], out_hbm.at[pl.ds(b * 8, 8), pl.ds(lt0, 2), :, :], wsems[s]
        )

    read_block(0, 0)
    read_block(1, 1)

    def pair_body(p, carry):
        for b in range(2):
            t = 2 * p + b
            pltpu.make_async_copy(
                src_hbm.at[pl.ds(0, _LB)], rows[b], rsems[b]
            ).wait()

            @pl.when(p > 0)
            def _():
                pltpu.make_async_copy(
                    blks[b], out_hbm.at[pl.ds(0, 8), pl.ds(0, 2), :, :],
                    wsems[b]
                ).wait()

            transpose_block(b)
            write_block(t, b)

            @pl.when(p < (_NT // 2 - 1))
            def _():
                read_block(t + 2, b)

        return carry

    lax.fori_loop(0, _NT // 2, pair_body, 0)
    pltpu.make_async_copy(blks[0], out_hbm.at[pl.ds(0, 8), pl.ds(0, 2), :, :],
                          wsems[0]).wait()
    pltpu.make_async_copy(blks[1], out_hbm.at[pl.ds(0, 8), pl.ds(0, 2), :, :],
                          wsems[1]).wait()


def kernel(x):
    x4 = (
        x.transpose(0, 2, 1)
        .reshape(_B, 8, 8, _L // 128, 128)
        .transpose(0, 1, 3, 2, 4)
        .reshape(512, _L // 128, 8, 128)
    )
    scat = jnp.asarray(_MSHUF)
    scratch = _scatter_pass(x4, scat)
    y4 = _untranspose_pass(scratch)
    return (
        y4.reshape(_B, 8, _L // 128, 8, 128)
        .transpose(0, 1, 3, 2, 4)
        .reshape(_B, _D, _L)
        .transpose(0, 2, 1)
    )


# speedup vs baseline: 3.4639x; 3.4639x over previous
import functools

import numpy as np
import jax
import jax.numpy as jnp
from jax import lax
from jax.experimental import pallas as pl
from jax.experimental.pallas import tpu as pltpu
from jax.experimental.pallas import tpu_sc as plsc

_B, _L, _D = 64, 2048, 64
_N = _B * _L

np.random.seed(0)
_MSHUF = np.random.permutation(np.arange(_N)).astype(np.int32)

_info = plsc.get_sparse_core_info()
_NC, _NS = _info.num_cores, _info.num_subcores
_NW = _NC * _NS
_RPW = _N // _NW
_LB = 256
_NT = _RPW // _LB
_TPB = _L // _LB

_mesh = plsc.VectorSubcoreMesh(core_axis_name="c", subcore_axis_name="s")
_PARAMS = pltpu.CompilerParams(use_tc_tiling_on_sc=False, needs_layout_passes=False)

def _lane_consts():
    iota = jnp.arange(16, dtype=jnp.int32)
    col = [iota + 16 * j for j in range(_LB // 16)]
    ltb = [jnp.full((16,), (16 * j) // 128, jnp.int32) for j in range(_LB // 16)]
    li = [iota + (16 * j) % 128 for j in range(_LB // 16)]
    return iota, col, ltb, li


@functools.partial(
    pl.kernel,
    mesh=_mesh,
    compiler_params=_PARAMS,
    out_type=jax.ShapeDtypeStruct((_N, _D), jnp.float32),
    scratch_types=[
        pltpu.VMEM((_NT, _LB), jnp.int32),
        pltpu.VMEM((8, 2, 8, 128), jnp.float32),
        pltpu.VMEM((8, 2, 8, 128), jnp.float32),
        pltpu.VMEM((_LB, _D), jnp.float32),
        pltpu.VMEM((_LB, _D), jnp.float32),
        pltpu.SemaphoreType.DMA,
        pltpu.SemaphoreType.DMA,
        pltpu.SemaphoreType.DMA,
        pltpu.SemaphoreType.DMA,
    ],
)
def _scatter_pass(x4_hbm, scat_hbm, out_hbm, sidx_v, blk0, blk1, rows0, rows1,
                  rsem0, rsem1, ssem0, ssem1):
    wid = lax.axis_index("s") * _NC + lax.axis_index("c")
    blks = (blk0, blk1)
    rows = (rows0, rows1)
    rsems = (rsem0, rsem1)
    ssems = (ssem0, ssem1)
    for t in range(_NT):
        pltpu.sync_copy(scat_hbm.at[pl.ds(wid * _RPW + t * _LB, _LB)],
                        sidx_v.at[t])
    _iota, col_ids, _ltb, _li = _lane_consts()

    def read_block(t, s):
        b = 2 * wid + (t // _TPB)
        lt0 = 2 * (t % _TPB)
        return pltpu.async_copy(
            x4_hbm.at[pl.ds(b * 8, 8), pl.ds(lt0, 2), :, :], blks[s], rsems[s]
        )

    def transpose_block(s):
        blk, row = blks[s], rows[s]

        @plsc.parallel_loop(0, _D, 1, unroll=2)
        def body(c):
            d_vec = (jnp.full((16,), c, jnp.int32) + _iota) & 63
            dtv = jnp.right_shift(d_vec, 3)
            div = jnp.bitwise_and(d_vec, 7)
            for j in range(_LB // 16):
                v = plsc.load_gather(blk, [dtv, _ltb[j], div, _li[j]])
                plsc.store_scatter(row, [col_ids[j], d_vec], v)

    def scatter_block(t, s):
        return pltpu.async_copy(rows[s], out_hbm.at[sidx_v.at[t]], ssems[s])

    read_block(0, 0)
    read_block(1, 1)

    def pair_body(p, carry):
        for b in range(2):
            t = 2 * p + b
            pltpu.make_async_copy(
                x4_hbm.at[pl.ds(0, 8), pl.ds(0, 2), :, :], blks[b], rsems[b]
            ).wait()

            @pl.when(p > 0)
            def _():
                pltpu.make_async_copy(rows[b], out_hbm.at[pl.ds(0, _LB)],
                                      ssems[b]).wait()

            transpose_block(b)
            scatter_block(t, b)

            @pl.when(p < (_NT // 2 - 1))
            def _():
                read_block(t + 2, b)

        return carry

    lax.fori_loop(0, _NT // 2, pair_body, 0)
    pltpu.make_async_copy(rows[0], out_hbm.at[pl.ds(0, _LB)], ssems[0]).wait()
    pltpu.make_async_copy(rows[1], out_hbm.at[pl.ds(0, _LB)], ssems[1]).wait()


@functools.partial(
    pl.kernel,
    mesh=_mesh,
    compiler_params=_PARAMS,
    out_type=jax.ShapeDtypeStruct((512, 16, 8, 128), jnp.float32),
    scratch_types=[
        pltpu.VMEM((_LB, _D), jnp.float32),
        pltpu.VMEM((_LB, _D), jnp.float32),
        pltpu.VMEM((8, 2, 8, 128), jnp.float32),
        pltpu.VMEM((8, 2, 8, 128), jnp.float32),
        pltpu.SemaphoreType.DMA,
        pltpu.SemaphoreType.DMA,
        pltpu.SemaphoreType.DMA,
        pltpu.SemaphoreType.DMA,
    ],
)
def _untranspose_pass(src_hbm, out_hbm, rows0, rows1, blk0, blk1,
                      rsem0, rsem1, wsem0, wsem1):
    wid = lax.axis_index("s") * _NC + lax.axis_index("c")
    rows = (rows0, rows1)
    blks = (blk0, blk1)
    rsems = (rsem0, rsem1)
    wsems = (wsem0, wsem1)
    base = wid * _RPW
    _iota, col_ids, _ltb, _li = _lane_consts()

    def read_block(t, s):
        return pltpu.async_copy(
            src_hbm.at[pl.ds(base + t * _LB, _LB)], rows[s], rsems[s]
        )

    def transpose_block(s):
        row, blk = rows[s], blks[s]

        @plsc.parallel_loop(0, _D, 1, unroll=2)
        def body(c):
            d_vec = (jnp.full((16,), c, jnp.int32) + _iota) & 63
            dtv = jnp.right_shift(d_vec, 3)
            div = jnp.bitwise_and(d_vec, 7)
            for j in range(_LB // 16):
                v = plsc.load_gather(row, [col_ids[j], d_vec])
                plsc.store_scatter(blk, [dtv, _ltb[j], div, _li[j]], v)

    def write_block(t, s):
        b = 2 * wid + (t // _TPB)
        lt0 = 2 * (t % _TPB)
        return pltpu.async_copy(
            blks[s], out_hbm.at[pl.ds(b * 8, 8), pl.ds(lt0, 2), :, :], wsems[s]
        )

    read_block(0, 0)
    read_block(1, 1)

    def pair_body(p, carry):
        for b in range(2):
            t = 2 * p + b
            pltpu.make_async_copy(
                src_hbm.at[pl.ds(0, _LB)], rows[b], rsems[b]
            ).wait()

            @pl.when(p > 0)
            def _():
                pltpu.make_async_copy(
                    blks[b], out_hbm.at[pl.ds(0, 8), pl.ds(0, 2), :, :],
                    wsems[b]
                ).wait()

            transpose_block(b)
            write_block(t, b)

            @pl.when(p < (_NT // 2 - 1))
            def _():
                read_block(t + 2, b)

        return carry

    lax.fori_loop(0, _NT // 2, pair_body, 0)
    pltpu.make_async_copy(blks[0], out_hbm.at[pl.ds(0, 8), pl.ds(0, 2), :, :],
                          wsems[0]).wait()
    pltpu.make_async_copy(blks[1], out_hbm.at[pl.ds(0, 8), pl.ds(0, 2), :, :],
                          wsems[1]).wait()


def kernel(x):
    x4 = (
        x.transpose(0, 2, 1)
        .reshape(_B, 8, 8, _L // 128, 128)
        .transpose(0, 1, 3, 2, 4)
        .reshape(512, _L // 128, 8, 128)
    )
    scat = jnp.asarray(_MSHUF)
    scratch = _scatter_pass(x4, scat)
    y4 = _untranspose_pass(scratch)
    return (
        y4.reshape(_B, 8, _L // 128, 8, 128)
        .transpose(0, 1, 3, 2, 4)
        .reshape(_B, _D, _L)
        .transpose(0, 2, 1)
    )


# speedup vs baseline: 3.7410x; 1.0800x over previous
import functools

import numpy as np
import jax
import jax.numpy as jnp
from jax import lax
from jax.experimental import pallas as pl
from jax.experimental.pallas import tpu as pltpu
from jax.experimental.pallas import tpu_sc as plsc

_B, _L, _D = 64, 2048, 64
_N = _B * _L

np.random.seed(0)
_MSHUF = np.random.permutation(np.arange(_N)).astype(np.int32)

_info = plsc.get_sparse_core_info()
_NC, _NS = _info.num_cores, _info.num_subcores
_NW = _NC * _NS
_RPW = _N // _NW
_LB = 256
_NT = _RPW // _LB
_TPB = _L // _LB

_mesh = plsc.VectorSubcoreMesh(core_axis_name="c", subcore_axis_name="s")
_PARAMS = pltpu.CompilerParams(
    use_tc_tiling_on_sc=False, needs_layout_passes=False, skip_device_barrier=True
)

def _lane_consts():
    iota = jnp.arange(16, dtype=jnp.int32)
    col = [iota + 16 * j for j in range(_LB // 16)]
    ltb = [jnp.full((16,), (16 * j) // 128, jnp.int32) for j in range(_LB // 16)]
    li = [iota + (16 * j) % 128 for j in range(_LB // 16)]
    return iota, col, ltb, li


@functools.partial(
    pl.kernel,
    mesh=_mesh,
    compiler_params=_PARAMS,
    out_type=jax.ShapeDtypeStruct((_N, _D), jnp.float32),
    scratch_types=[
        pltpu.VMEM((_N // _NW,), jnp.int32),
        pltpu.VMEM((8, 2, 8, 128), jnp.float32),
        pltpu.VMEM((8, 2, 8, 128), jnp.float32),
        pltpu.VMEM((_LB, _D), jnp.float32),
        pltpu.VMEM((_LB, _D), jnp.float32),
        pltpu.SemaphoreType.DMA,
        pltpu.SemaphoreType.DMA,
        pltpu.SemaphoreType.DMA,
        pltpu.SemaphoreType.DMA,
    ],
)
def _scatter_pass(x4_hbm, scat_hbm, out_hbm, sidx_v, blk0, blk1, rows0, rows1,
                  rsem0, rsem1, ssem0, ssem1):
    wid = lax.axis_index("s") * _NC + lax.axis_index("c")
    blks = (blk0, blk1)
    rows = (rows0, rows1)
    rsems = (rsem0, rsem1)
    ssems = (ssem0, ssem1)
    pltpu.sync_copy(scat_hbm.at[pl.ds(wid * _RPW, _RPW)], sidx_v)
    _iota, col_ids, _ltb, _li = _lane_consts()

    def read_block(t, s):
        b = 2 * wid + (t // _TPB)
        lt0 = 2 * (t % _TPB)
        return pltpu.async_copy(
            x4_hbm.at[pl.ds(b * 8, 8), pl.ds(lt0, 2), :, :], blks[s], rsems[s]
        )

    def transpose_block(s):
        blk, row = blks[s], rows[s]

        @plsc.parallel_loop(0, _D, 1, unroll=2)
        def body(c):
            d_vec = (jnp.full((16,), c, jnp.int32) + _iota) & 63
            dtv = jnp.right_shift(d_vec, 3)
            div = jnp.bitwise_and(d_vec, 7)
            for j in range(_LB // 16):
                v = plsc.load_gather(blk, [dtv, _ltb[j], div, _li[j]])
                plsc.store_scatter(row, [col_ids[j], d_vec], v)

    def scatter_block(t, s):
        return pltpu.async_copy(
            rows[s], out_hbm.at[sidx_v.at[pl.ds(t * _LB, _LB)]], ssems[s]
        )

    read_block(0, 0)
    read_block(1, 1)

    def pair_body(p, carry):
        for b in range(2):
            t = 2 * p + b
            pltpu.make_async_copy(
                x4_hbm.at[pl.ds(0, 8), pl.ds(0, 2), :, :], blks[b], rsems[b]
            ).wait()

            @pl.when(p > 0)
            def _():
                pltpu.make_async_copy(rows[b], out_hbm.at[pl.ds(0, _LB)],
                                      ssems[b]).wait()

            transpose_block(b)
            scatter_block(t, b)

            @pl.when(p < (_NT // 2 - 1))
            def _():
                read_block(t + 2, b)

        return carry

    lax.fori_loop(0, _NT // 2, pair_body, 0)
    pltpu.make_async_copy(rows[0], out_hbm.at[pl.ds(0, _LB)], ssems[0]).wait()
    pltpu.make_async_copy(rows[1], out_hbm.at[pl.ds(0, _LB)], ssems[1]).wait()


@functools.partial(
    pl.kernel,
    mesh=_mesh,
    compiler_params=_PARAMS,
    out_type=jax.ShapeDtypeStruct((512, 16, 8, 128), jnp.float32),
    scratch_types=[
        pltpu.VMEM((_LB, _D), jnp.float32),
        pltpu.VMEM((_LB, _D), jnp.float32),
        pltpu.VMEM((8, 2, 8, 128), jnp.float32),
        pltpu.VMEM((8, 2, 8, 128), jnp.float32),
        pltpu.SemaphoreType.DMA,
        pltpu.SemaphoreType.DMA,
        pltpu.SemaphoreType.DMA,
        pltpu.SemaphoreType.DMA,
    ],
)
def _untranspose_pass(src_hbm, out_hbm, rows0, rows1, blk0, blk1,
                      rsem0, rsem1, wsem0, wsem1):
    wid = lax.axis_index("s") * _NC + lax.axis_index("c")
    rows = (rows0, rows1)
    blks = (blk0, blk1)
    rsems = (rsem0, rsem1)
    wsems = (wsem0, wsem1)
    base = wid * _RPW
    _iota, col_ids, _ltb, _li = _lane_consts()

    def read_block(t, s):
        return pltpu.async_copy(
            src_hbm.at[pl.ds(base + t * _LB, _LB)], rows[s], rsems[s]
        )

    def transpose_block(s):
        row, blk = rows[s], blks[s]

        @plsc.parallel_loop(0, _D, 1, unroll=2)
        def body(c):
            d_vec = (jnp.full((16,), c, jnp.int32) + _iota) & 63
            dtv = jnp.right_shift(d_vec, 3)
            div = jnp.bitwise_and(d_vec, 7)
            for j in range(_LB // 16):
                v = plsc.load_gather(row, [col_ids[j], d_vec])
                plsc.store_scatter(blk, [dtv, _ltb[j], div, _li[j]], v)

    def write_block(t, s):
        b = 2 * wid + (t // _TPB)
        lt0 = 2 * (t % _TPB)
        return pltpu.async_copy(
            blks[s], out_hbm.at[pl.ds(b * 8, 8), pl.ds(lt0, 2), :, :], wsems[s]
        )

    read_block(0, 0)
    read_block(1, 1)

    def pair_body(p, carry):
        for b in range(2):
            t = 2 * p + b
            pltpu.make_async_copy(
                src_hbm.at[pl.ds(0, _LB)], rows[b], rsems[b]
            ).wait()

            @pl.when(p > 0)
            def _():
                pltpu.make_async_copy(
                    blks[b], out_hbm.at[pl.ds(0, 8), pl.ds(0, 2), :, :],
                    wsems[b]
                ).wait()

            transpose_block(b)
            write_block(t, b)

            @pl.when(p < (_NT // 2 - 1))
            def _():
                read_block(t + 2, b)

        return carry

    lax.fori_loop(0, _NT // 2, pair_body, 0)
    pltpu.make_async_copy(blks[0], out_hbm.at[pl.ds(0, 8), pl.ds(0, 2), :, :],
                          wsems[0]).wait()
    pltpu.make_async_copy(blks[1], out_hbm.at[pl.ds(0, 8), pl.ds(0, 2), :, :],
                          wsems[1]).wait()


def kernel(x):
    x4 = (
        x.transpose(0, 2, 1)
        .reshape(_B, 8, 8, _L // 128, 128)
        .transpose(0, 1, 3, 2, 4)
        .reshape(512, _L // 128, 8, 128)
    )
    scat = jnp.asarray(_MSHUF)
    scratch = _scatter_pass(x4, scat)
    y4 = _untranspose_pass(scratch)
    return (
        y4.reshape(_B, 8, _L // 128, 8, 128)
        .transpose(0, 1, 3, 2, 4)
        .reshape(_B, _D, _L)
        .transpose(0, 2, 1)
    )
